# Initial kernel scaffold; baseline (speedup 1.0000x reference)
import jax
import jax.numpy as jnp
from jax.experimental import pallas as pl


def kernel(node, edge_index):
    raise NotImplementedError("write your pallas kernel here")


# speedup vs baseline: 5.2758x; 5.2758x over previous
import functools

import jax
import jax.numpy as jnp
from jax import lax
from jax.experimental import pallas as pl
from jax.experimental.pallas import tpu as pltpu
from jax.experimental.pallas import tpu_sc as plsc

N_DST = 10000
D = 128
NC = 2
NS = 16
NW = NC * NS
CHUNK = 80
ACC_ROWS = 10240
ROWS_PER_TILE = ACC_ROWS // NS


@functools.partial(jax.jit, static_argnums=())
def _sc_partial_sums(node, src, dst, zeros):
    E = src.shape[0]
    per_tile = E // NW
    n_chunks = per_tile // CHUNK
    assert per_tile % CHUNK == 0 and E % NW == 0

    mesh = plsc.VectorSubcoreMesh(core_axis_name="c", subcore_axis_name="s")

    @functools.partial(
        pl.kernel,
        mesh=mesh,
        out_type=jax.ShapeDtypeStruct((NC * ACC_ROWS, D), jnp.float32),
        scratch_types=[
            pltpu.VMEM((CHUNK,), jnp.int32),
            pltpu.VMEM((CHUNK,), jnp.int32),
            pltpu.VMEM((CHUNK, D), jnp.float32),
            pltpu.VMEM_SHARED((ACC_ROWS, D), jnp.float32),
            pltpu.SemaphoreType.DMA,
        ],
    )
    def k(node_hbm, src_hbm, dst_hbm, zeros_hbm, out_hbm,
          src_v, dst_v, rows_v, acc, sem):
        c = lax.axis_index("c")
        s = lax.axis_index("s")
        wid = s * NC + c

        r0 = s * ROWS_PER_TILE
        pltpu.sync_copy(zeros_hbm.at[pl.ds(r0, ROWS_PER_TILE)],
                        acc.at[pl.ds(r0, ROWS_PER_TILE)])
        plsc.subcore_barrier()

        base0 = wid * per_tile

        def body(i, carry):
            base = base0 + i * CHUNK
            pltpu.sync_copy(src_hbm.at[pl.ds(base, CHUNK)], src_v)
            pltpu.sync_copy(dst_hbm.at[pl.ds(base, CHUNK)], dst_v)
            pltpu.async_copy(node_hbm.at[src_v], rows_v, sem).wait()
            pltpu.sync_copy(rows_v, acc.at[dst_v], add=True)
            return carry

        lax.fori_loop(0, n_chunks, body, 0)
        plsc.subcore_barrier()

        pltpu.sync_copy(acc.at[pl.ds(r0, ROWS_PER_TILE)],
                        out_hbm.at[pl.ds(c * ACC_ROWS + r0, ROWS_PER_TILE)])

    return k(node, src, dst, zeros)


def _combine(partials):
    R = 400

    def body(a_ref, b_ref, o_ref):
        o_ref[...] = a_ref[...] + b_ref[...]

    return pl.pallas_call(
        body,
        grid=(N_DST // R,),
        in_specs=[pl.BlockSpec((R, D), lambda i: (i, 0)),
                  pl.BlockSpec((R, D), lambda i: (i, 0))],
        out_specs=pl.BlockSpec((R, D), lambda i: (i, 0)),
        out_shape=jax.ShapeDtypeStruct((N_DST, D), jnp.float32),
    )(partials[:N_DST], partials[ACC_ROWS:ACC_ROWS + N_DST])


def kernel(node, edge_index):
    ei = edge_index.astype(jnp.int32)
    zeros = jnp.zeros((ACC_ROWS, D), jnp.float32)
    partials = _sc_partial_sums(node, ei[0], ei[1], zeros)
    return _combine(partials)
